# baseline (speedup 1.0000x reference)
import functools

import jax
import jax.numpy as jnp
from jax import lax
from jax.experimental import pallas as pl
from jax.experimental.pallas import tpu as pltpu
from jax.experimental.pallas import tpu_sc as plsc

B = 16384
EMB = 128
NC = 2
NS = 16
NW = NC * NS
BPW = B // NW
CH = 128
NCH = BPW // CH
GRP = CH // 16


def _mf_body(u_id, i_id, user_emb, user_bias, item_emb, item_bias, mean, out,
             uidx, iidx, urows, irows, bu, bi, mv, mat, outv, sem_r, sem_b):
    c = lax.axis_index("c")
    s = lax.axis_index("s")
    wid = s * NC + c
    base = pl.multiple_of(wid * BPW, BPW)

    pltpu.sync_copy(u_id.at[pl.ds(base, BPW)], uidx)
    pltpu.sync_copy(i_id.at[pl.ds(base, BPW)], iidx)
    pltpu.sync_copy(mean, mv.at[pl.ds(0, 1)])

    cp_bu = pltpu.async_copy(user_bias.at[uidx], bu, sem_b)
    cp_bi = pltpu.async_copy(item_bias.at[iidx], bi, sem_b)

    lane = lax.iota(jnp.int32, 16)
    mean_s = mv[pl.ds(0, 16)][0]

    cp_bu.wait()
    cp_bi.wait()

    for ci in range(NCH):
        coff = ci * CH
        cp_u = pltpu.async_copy(
            user_emb.at[uidx.at[pl.ds(coff, CH)]], urows, sem_r)
        cp_i = pltpu.async_copy(
            item_emb.at[iidx.at[pl.ds(coff, CH)]], irows, sem_r)
        cp_u.wait()
        cp_i.wait()

        def group(g, _, coff=coff):
            row0 = pl.multiple_of(g * 16, 16)
            for j in range(16):
                r = row0 + j
                acc = urows[r, pl.ds(0, 16)] * irows[r, pl.ds(0, 16)]
                for v in range(1, 8):
                    acc = acc + urows[r, pl.ds(16 * v, 16)] * irows[r, pl.ds(16 * v, 16)]
                plsc.store_scatter(mat, [lane, jnp.full((16,), j, jnp.int32)], acc)
            tot = mat[0, :]
            for l in range(1, 16):
                tot = tot + mat[l, :]
            off = pl.multiple_of(coff + row0, 16)
            outv[pl.ds(off, 16)] = (
                tot + bu[pl.ds(off, 16)] + bi[pl.ds(off, 16)] + mean_s)
            return 0

        lax.fori_loop(0, GRP, group, 0)

    pltpu.sync_copy(outv, out.at[pl.ds(base, BPW)])


@functools.partial(jax.jit, donate_argnums=())
def _mf(u_id, i_id, user_emb, user_bias, item_emb, item_bias, mean):
    mesh = plsc.VectorSubcoreMesh(core_axis_name="c", subcore_axis_name="s")
    k = pl.kernel(
        _mf_body,
        mesh=mesh,
        compiler_params=pltpu.CompilerParams(needs_layout_passes=False),
        out_type=jax.ShapeDtypeStruct((B,), jnp.float32),
        scratch_types=[
            pltpu.VMEM((BPW,), jnp.int32),
            pltpu.VMEM((BPW,), jnp.int32),
            pltpu.VMEM((CH, EMB), jnp.float32),
            pltpu.VMEM((CH, EMB), jnp.float32),
            pltpu.VMEM((BPW,), jnp.float32),
            pltpu.VMEM((BPW,), jnp.float32),
            pltpu.VMEM((16,), jnp.float32),
            pltpu.VMEM((16, 16), jnp.float32),
            pltpu.VMEM((BPW,), jnp.float32),
            pltpu.SemaphoreType.DMA,
            pltpu.SemaphoreType.DMA,
        ],
    )
    return k(u_id, i_id, user_emb, user_bias, item_emb, item_bias, mean)


def kernel(u_id, i_id, user_emb, user_bias, item_emb, item_bias, mean):
    return _mf(u_id, i_id, user_emb, user_bias.reshape(-1), item_emb,
               item_bias.reshape(-1), mean)


# speedup vs baseline: 1.0988x; 1.0988x over previous
import functools

import jax
import jax.numpy as jnp
from jax import lax
from jax.experimental import pallas as pl
from jax.experimental.pallas import tpu as pltpu
from jax.experimental.pallas import tpu_sc as plsc

B = 16384
EMB = 128
NC = 2
NS = 16
NW = NC * NS
BPW = B // NW
CH = 128
NCH = BPW // CH
GRP = CH // 16


def _mf_body(u_id, i_id, user_emb, user_bias, item_emb, item_bias, mean, out,
             uidx, iidx, urows0, irows0, urows1, irows1, bu, bi, mv, mat, outv,
             sem0, sem1, sem_b):
    c = lax.axis_index("c")
    s = lax.axis_index("s")
    wid = s * NC + c
    base = pl.multiple_of(wid * BPW, BPW)

    pltpu.sync_copy(u_id.at[pl.ds(base, BPW)], uidx)
    pltpu.sync_copy(i_id.at[pl.ds(base, BPW)], iidx)
    pltpu.sync_copy(mean, mv.at[pl.ds(0, 1)])

    cp_bu = pltpu.async_copy(user_bias.at[uidx], bu, sem_b)
    cp_bi = pltpu.async_copy(item_bias.at[iidx], bi, sem_b)

    lane = lax.iota(jnp.int32, 16)
    mean_s = mv[pl.ds(0, 16)][0]

    bufs = [(urows0, irows0, sem0), (urows1, irows1, sem1)]

    def start(ci):
        ub, ib, sem = bufs[ci % 2]
        coff = ci * CH
        cu = pltpu.async_copy(user_emb.at[uidx.at[pl.ds(coff, CH)]], ub, sem)
        cv = pltpu.async_copy(item_emb.at[iidx.at[pl.ds(coff, CH)]], ib, sem)
        return cu, cv

    pend = start(0)
    cp_bu.wait()
    cp_bi.wait()

    for ci in range(NCH):
        nxt = start(ci + 1) if ci + 1 < NCH else None
        pend[0].wait()
        pend[1].wait()
        ub, ib, _ = bufs[ci % 2]
        coff = ci * CH

        def group(g, _, ub=ub, ib=ib, coff=coff):
            row0 = pl.multiple_of(g * 16, 16)
            for j in range(16):
                r = row0 + j
                acc = ub[r, pl.ds(0, 16)] * ib[r, pl.ds(0, 16)]
                for v in range(1, 8):
                    acc = acc + ub[r, pl.ds(16 * v, 16)] * ib[r, pl.ds(16 * v, 16)]
                plsc.store_scatter(mat, [lane, jnp.full((16,), j, jnp.int32)], acc)
            tot = mat[0, :]
            for l in range(1, 16):
                tot = tot + mat[l, :]
            off = pl.multiple_of(coff + row0, 16)
            outv[pl.ds(off, 16)] = (
                tot + bu[pl.ds(off, 16)] + bi[pl.ds(off, 16)] + mean_s)
            return 0

        lax.fori_loop(0, GRP, group, 0)
        pend = nxt

    pltpu.sync_copy(outv, out.at[pl.ds(base, BPW)])


@functools.partial(jax.jit, donate_argnums=())
def _mf(u_id, i_id, user_emb, user_bias, item_emb, item_bias, mean):
    mesh = plsc.VectorSubcoreMesh(core_axis_name="c", subcore_axis_name="s")
    k = pl.kernel(
        _mf_body,
        mesh=mesh,
        compiler_params=pltpu.CompilerParams(needs_layout_passes=False),
        out_type=jax.ShapeDtypeStruct((B,), jnp.float32),
        scratch_types=[
            pltpu.VMEM((BPW,), jnp.int32),
            pltpu.VMEM((BPW,), jnp.int32),
            pltpu.VMEM((CH, EMB), jnp.float32),
            pltpu.VMEM((CH, EMB), jnp.float32),
            pltpu.VMEM((CH, EMB), jnp.float32),
            pltpu.VMEM((CH, EMB), jnp.float32),
            pltpu.VMEM((BPW,), jnp.float32),
            pltpu.VMEM((BPW,), jnp.float32),
            pltpu.VMEM((16,), jnp.float32),
            pltpu.VMEM((16, 16), jnp.float32),
            pltpu.VMEM((BPW,), jnp.float32),
            pltpu.SemaphoreType.DMA,
            pltpu.SemaphoreType.DMA,
            pltpu.SemaphoreType.DMA,
        ],
    )
    return k(u_id, i_id, user_emb, user_bias, item_emb, item_bias, mean)


def kernel(u_id, i_id, user_emb, user_bias, item_emb, item_bias, mean):
    return _mf(u_id, i_id, user_emb, user_bias.reshape(-1), item_emb,
               item_bias.reshape(-1), mean)
